# Initial kernel scaffold; baseline (speedup 1.0000x reference)
import jax
import jax.numpy as jnp
from jax.experimental import pallas as pl


def kernel(z, A):
    raise NotImplementedError("write your pallas kernel here")


# speedup vs baseline: 7.6718x; 7.6718x over previous
import functools

import numpy as np
import jax
import jax.numpy as jnp
from jax.experimental import pallas as pl
from jax.experimental.pallas import tpu as pltpu

_KNOTS = 17
_DT = 3.0 / (_KNOTS - 1)
_NK = _KNOTS - 1


def _knot_consts():
    k = np.arange(1, _KNOTS, dtype=np.float64)
    t = k * (3.0 / (_KNOTS - 1))
    phi = np.exp(-t * t / 2.0)
    w = np.full(_NK, 2.0 * _DT, dtype=np.float64)
    w[-1] = _DT
    return phi.astype(np.float32), (w * phi).astype(np.float32)


_PHI, _WPHI = _knot_consts()


def _sigreg_kernel(z_ref, a_ref, o_ref, acc_c, acc_s, *, nb, n_total, rc):
    ni = pl.program_id(1)

    @pl.when(ni == 0)
    def _():
        acc_c[...] = jnp.zeros_like(acc_c)
        acc_s[...] = jnp.zeros_like(acc_s)

    a = a_ref[...]
    inv = 1.0 / jnp.maximum(jnp.sqrt(jnp.sum(a * a, axis=0)), 1e-8)
    an = a * inv[None, :]

    bn = z_ref.shape[0]
    bp = a_ref.shape[1]
    sums_c = [None] * _NK
    sums_s = [None] * _NK
    for c0 in range(0, bn, rc):
        zc = z_ref[c0:c0 + rc, :]
        proj = jnp.dot(zc, an, preferred_element_type=jnp.float32)
        theta = proj * _DT
        c1 = jnp.cos(theta)
        s1 = jnp.sin(theta)
        ck, sk = c1, s1
        for k in range(_NK):
            pc = jnp.sum(ck, axis=0, keepdims=True)
            ps = jnp.sum(sk, axis=0, keepdims=True)
            sums_c[k] = pc if sums_c[k] is None else sums_c[k] + pc
            sums_s[k] = ps if sums_s[k] is None else sums_s[k] + ps
            if k < _NK - 1:
                ck, sk = ck * c1 - sk * s1, sk * c1 + ck * s1

    acc_c[...] += jnp.concatenate(sums_c, axis=0)
    acc_s[...] += jnp.concatenate(sums_s, axis=0)

    @pl.when(ni == nb - 1)
    def _():
        inv_n = np.float32(1.0 / n_total)
        cm = acc_c[...] * inv_n
        sm = acc_s[...] * inv_n
        k = jax.lax.broadcasted_iota(jnp.int32, (_NK, 1), 0).astype(jnp.float32) + 1.0
        t = k * np.float32(_DT)
        phi = jnp.exp(-0.5 * t * t)
        w = jnp.where(k == _NK, np.float32(_DT), np.float32(2.0 * _DT))
        wphi = w * phi
        err = (cm - phi) ** 2 + sm ** 2
        stat = jnp.sum(err * wphi, axis=0) * np.float32(n_total)
        o_ref[0, :] = stat


def kernel(z, A):
    n, d = z.shape
    d2, p = A.shape
    assert d == d2
    bp = min(512, p)
    bn = min(512, n)
    rc = min(128, bn)
    nb = n // bn
    assert n % bn == 0 and p % bp == 0

    stat = pl.pallas_call(
        functools.partial(_sigreg_kernel, nb=nb, n_total=n, rc=rc),
        out_shape=jax.ShapeDtypeStruct((1, p), jnp.float32),
        grid=(p // bp, nb),
        in_specs=[
            pl.BlockSpec((bn, d), lambda pi, ni: (ni, 0)),
            pl.BlockSpec((d, bp), lambda pi, ni: (0, pi)),
        ],
        out_specs=pl.BlockSpec((1, bp), lambda pi, ni: (0, pi)),
        scratch_shapes=[
            pltpu.VMEM((_NK, bp), jnp.float32),
            pltpu.VMEM((_NK, bp), jnp.float32),
        ],
        compiler_params=pltpu.CompilerParams(
            dimension_semantics=("parallel", "arbitrary"),
        ),
        name="sigreg_ecf",
    )(z, A)
    return jnp.mean(stat)


# speedup vs baseline: 13.0042x; 1.6951x over previous
import functools

import numpy as np
import jax
import jax.numpy as jnp
from jax.experimental import pallas as pl
from jax.experimental.pallas import tpu as pltpu

_KNOTS = 17
_DT = np.float32(3.0 / (_KNOTS - 1))
_NK = _KNOTS - 1

_TWO_OVER_PI = np.float32(2.0 / np.pi)


def _split_pio2():
    hi = np.float32(np.pi / 2)
    bits = np.frombuffer(hi.tobytes(), dtype=np.uint32)[0]
    hi_trunc = np.frombuffer(
        np.uint32(bits & np.uint32(0xFFFFF000)).tobytes(), dtype=np.float32
    )[0]
    lo = np.float32(np.pi / 2 - np.float64(hi_trunc))
    return np.float32(hi_trunc), lo


_PIO2_C1, _PIO2_C2 = _split_pio2()

_S1 = np.float32(-1.6666654611e-01)
_S2 = np.float32(8.3321608736e-03)
_S3 = np.float32(-1.9515295891e-04)
_C1F = np.float32(4.1666667908e-02)
_C2F = np.float32(-1.3888889225e-03)
_C3F = np.float32(2.4760126784e-05)


def _sincos(x):
    nf = jax.lax.round(x * _TWO_OVER_PI,
                       jax.lax.RoundingMethod.TO_NEAREST_EVEN)
    q = nf.astype(jnp.int32)
    r = x - nf * _PIO2_C1
    r = r - nf * _PIO2_C2
    s = r * r
    ps = _S3
    ps = ps * s + _S2
    ps = ps * s + _S1
    ps = ps * s + np.float32(1.0)
    sinr = ps * r
    pc = _C3F
    pc = pc * s + _C2F
    pc = pc * s + _C1F
    cosr = (np.float32(1.0) - np.float32(0.5) * s) + (s * s) * pc
    swap = (q & 1) == 1
    cos_sel = jnp.where(swap, sinr, cosr)
    sin_sel = jnp.where(swap, cosr, sinr)
    sc = ((q + 1) & 2) << 30
    ss = (q & 2) << 30
    cos_out = jax.lax.bitcast_convert_type(
        jax.lax.bitcast_convert_type(cos_sel, jnp.int32) ^ sc, jnp.float32)
    sin_out = jax.lax.bitcast_convert_type(
        jax.lax.bitcast_convert_type(sin_sel, jnp.int32) ^ ss, jnp.float32)
    return cos_out, sin_out


def _reduce8(x):
    r = x.shape[0]
    acc = x[0:8, :]
    for i in range(8, r, 8):
        acc = acc + x[i:i + 8, :]
    return acc


def _sigreg_kernel(z_ref, a_ref, o_ref, acc_c, acc_s, *, nb, n_total, rc):
    ni = pl.program_id(1)

    @pl.when(ni == 0)
    def _():
        acc_c[...] = jnp.zeros_like(acc_c)
        acc_s[...] = jnp.zeros_like(acc_s)

    a = a_ref[...]
    inv = 1.0 / jnp.maximum(jnp.sqrt(jnp.sum(a * a, axis=0)), 1e-8)
    an = a * inv[None, :]

    bn = z_ref.shape[0]
    for c0 in range(0, bn, rc):
        zc = z_ref[c0:c0 + rc, :]
        proj = jnp.dot(zc, an, preferred_element_type=jnp.float32)
        theta = proj * _DT
        c1, s1 = _sincos(theta)
        tc = c1 + c1
        acc_c[0:8, :] += _reduce8(c1)
        acc_s[0:8, :] += _reduce8(s1)
        ckm, sk_m = c1, s1
        ck = tc * c1 - np.float32(1.0)
        sk = tc * s1
        acc_c[8:16, :] += _reduce8(ck)
        acc_s[8:16, :] += _reduce8(sk)
        for k in range(3, _NK + 1):
            ck, ckm = tc * ck - ckm, ck
            sk, sk_m = tc * sk - sk_m, sk
            r0 = (k - 1) * 8
            acc_c[r0:r0 + 8, :] += _reduce8(ck)
            acc_s[r0:r0 + 8, :] += _reduce8(sk)

    @pl.when(ni == nb - 1)
    def _():
        inv_n = np.float32(1.0 / n_total)
        cm = jnp.concatenate(
            [jnp.sum(acc_c[8 * k:8 * k + 8, :], axis=0, keepdims=True)
             for k in range(_NK)], axis=0) * inv_n
        sm = jnp.concatenate(
            [jnp.sum(acc_s[8 * k:8 * k + 8, :], axis=0, keepdims=True)
             for k in range(_NK)], axis=0) * inv_n
        k = jax.lax.broadcasted_iota(jnp.int32, (_NK, 1), 0).astype(jnp.float32) + 1.0
        t = k * _DT
        phi = jnp.exp(-0.5 * t * t)
        w = jnp.where(k == _NK, _DT, np.float32(2.0) * _DT)
        wphi = w * phi
        err = (cm - phi) ** 2 + sm ** 2
        stat = jnp.sum(err * wphi, axis=0) * np.float32(n_total)
        o_ref[0, :] = stat


def kernel(z, A):
    n, d = z.shape
    d2, p = A.shape
    assert d == d2
    bp = min(512, p)
    bn = min(512, n)
    rc = min(64, bn)
    nb = n // bn
    assert n % bn == 0 and p % bp == 0

    stat = pl.pallas_call(
        functools.partial(_sigreg_kernel, nb=nb, n_total=n, rc=rc),
        out_shape=jax.ShapeDtypeStruct((1, p), jnp.float32),
        grid=(p // bp, nb),
        in_specs=[
            pl.BlockSpec((bn, d), lambda pi, ni: (ni, 0)),
            pl.BlockSpec((d, bp), lambda pi, ni: (0, pi)),
        ],
        out_specs=pl.BlockSpec((1, bp), lambda pi, ni: (0, pi)),
        scratch_shapes=[
            pltpu.VMEM((8 * _NK, bp), jnp.float32),
            pltpu.VMEM((8 * _NK, bp), jnp.float32),
        ],
        compiler_params=pltpu.CompilerParams(
            dimension_semantics=("parallel", "arbitrary"),
        ),
        name="sigreg_ecf",
    )(z, A)
    return jnp.mean(stat)


# speedup vs baseline: 13.2423x; 1.0183x over previous
import functools

import numpy as np
import jax
import jax.numpy as jnp
from jax.experimental import pallas as pl
from jax.experimental.pallas import tpu as pltpu

_BP = 512
_BN = 512
_RC = 64

_KNOTS = 17
_DT = np.float32(3.0 / (_KNOTS - 1))
_NK = _KNOTS - 1

_TWO_OVER_PI = np.float32(2.0 / np.pi)


def _split_pio2():
    hi = np.float32(np.pi / 2)
    bits = np.frombuffer(hi.tobytes(), dtype=np.uint32)[0]
    hi_trunc = np.frombuffer(
        np.uint32(bits & np.uint32(0xFFFFF000)).tobytes(), dtype=np.float32
    )[0]
    lo = np.float32(np.pi / 2 - np.float64(hi_trunc))
    return np.float32(hi_trunc), lo


_PIO2_C1, _PIO2_C2 = _split_pio2()

_S1 = np.float32(-1.6666654611e-01)
_S2 = np.float32(8.3321608736e-03)
_S3 = np.float32(-1.9515295891e-04)
_C1F = np.float32(4.1666667908e-02)
_C2F = np.float32(-1.3888889225e-03)
_C3F = np.float32(2.4760126784e-05)


def _sincos(x):
    nf = jax.lax.round(x * _TWO_OVER_PI,
                       jax.lax.RoundingMethod.TO_NEAREST_EVEN)
    q = nf.astype(jnp.int32)
    r = x - nf * _PIO2_C1
    r = r - nf * _PIO2_C2
    s = r * r
    ps = _S3
    ps = ps * s + _S2
    ps = ps * s + _S1
    ps = ps * s + np.float32(1.0)
    sinr = ps * r
    pc = _C3F
    pc = pc * s + _C2F
    pc = pc * s + _C1F
    cosr = (np.float32(1.0) - np.float32(0.5) * s) + (s * s) * pc
    swap = (q & 1) == 1
    cos_sel = jnp.where(swap, sinr, cosr)
    sin_sel = jnp.where(swap, cosr, sinr)
    sc = ((q + 1) & 2) << 30
    ss = (q & 2) << 30
    cos_out = jax.lax.bitcast_convert_type(
        jax.lax.bitcast_convert_type(cos_sel, jnp.int32) ^ sc, jnp.float32)
    sin_out = jax.lax.bitcast_convert_type(
        jax.lax.bitcast_convert_type(sin_sel, jnp.int32) ^ ss, jnp.float32)
    return cos_out, sin_out


def _reduce8(x):
    r = x.shape[0]
    acc = x[0:8, :]
    for i in range(8, r, 8):
        acc = acc + x[i:i + 8, :]
    return acc


def _sigreg_kernel(z_ref, a_ref, o_ref, acc_c, acc_s, an_s, *, nb, n_total, rc):
    ni = pl.program_id(1)

    @pl.when(ni == 0)
    def _():
        acc_c[...] = jnp.zeros_like(acc_c)
        acc_s[...] = jnp.zeros_like(acc_s)
        a0 = a_ref[...]
        inv = 1.0 / jnp.maximum(
            jnp.sqrt(jnp.sum(a0 * a0, axis=0, keepdims=True)), 1e-8)
        an_s[...] = a0 * inv

    an = an_s[...]

    bn = z_ref.shape[0]
    for c0 in range(0, bn, rc):
        zc = z_ref[c0:c0 + rc, :]
        proj = jnp.dot(zc, an, preferred_element_type=jnp.float32)
        theta = proj * _DT
        c1, s1 = _sincos(theta)
        tc = c1 + c1
        acc_c[0:8, :] += _reduce8(c1)
        acc_s[0:8, :] += _reduce8(s1)
        ckm, sk_m = c1, s1
        ck = tc * c1 - np.float32(1.0)
        sk = tc * s1
        acc_c[8:16, :] += _reduce8(ck)
        acc_s[8:16, :] += _reduce8(sk)
        for k in range(3, _NK + 1):
            ck, ckm = tc * ck - ckm, ck
            sk, sk_m = tc * sk - sk_m, sk
            r0 = (k - 1) * 8
            acc_c[r0:r0 + 8, :] += _reduce8(ck)
            acc_s[r0:r0 + 8, :] += _reduce8(sk)

    @pl.when(ni == nb - 1)
    def _():
        inv_n = np.float32(1.0 / n_total)
        cm = jnp.concatenate(
            [jnp.sum(acc_c[8 * k:8 * k + 8, :], axis=0, keepdims=True)
             for k in range(_NK)], axis=0) * inv_n
        sm = jnp.concatenate(
            [jnp.sum(acc_s[8 * k:8 * k + 8, :], axis=0, keepdims=True)
             for k in range(_NK)], axis=0) * inv_n
        k = jax.lax.broadcasted_iota(jnp.int32, (_NK, 1), 0).astype(jnp.float32) + 1.0
        t = k * _DT
        phi = jnp.exp(-0.5 * t * t)
        w = jnp.where(k == _NK, _DT, np.float32(2.0) * _DT)
        wphi = w * phi
        err = (cm - phi) ** 2 + sm ** 2
        stat = jnp.sum(err * wphi, axis=0) * np.float32(n_total)
        o_ref[0, :] = stat


def kernel(z, A):
    n, d = z.shape
    d2, p = A.shape
    assert d == d2
    bp = min(_BP, p)
    bn = min(_BN, n)
    rc = min(_RC, bn)
    nb = n // bn
    assert n % bn == 0 and p % bp == 0

    stat = pl.pallas_call(
        functools.partial(_sigreg_kernel, nb=nb, n_total=n, rc=rc),
        out_shape=jax.ShapeDtypeStruct((1, p), jnp.float32),
        grid=(p // bp, nb),
        in_specs=[
            pl.BlockSpec((bn, d), lambda pi, ni: (ni, 0)),
            pl.BlockSpec((d, bp), lambda pi, ni: (0, pi)),
        ],
        out_specs=pl.BlockSpec((1, bp), lambda pi, ni: (0, pi)),
        scratch_shapes=[
            pltpu.VMEM((8 * _NK, bp), jnp.float32),
            pltpu.VMEM((8 * _NK, bp), jnp.float32),
            pltpu.VMEM((d, bp), jnp.float32),
        ],
        compiler_params=pltpu.CompilerParams(
            dimension_semantics=("parallel", "arbitrary"),
        ),
        name="sigreg_ecf",
    )(z, A)
    return jnp.mean(stat)


# speedup vs baseline: 13.5678x; 1.0246x over previous
import functools

import numpy as np
import jax
import jax.numpy as jnp
from jax.experimental import pallas as pl
from jax.experimental.pallas import tpu as pltpu

_BP = 512
_BN = 512
_RC = 128

_KNOTS = 17
_DT = np.float32(3.0 / (_KNOTS - 1))
_NK = _KNOTS - 1

_TWO_OVER_PI = np.float32(2.0 / np.pi)


def _split_pio2():
    hi = np.float32(np.pi / 2)
    bits = np.frombuffer(hi.tobytes(), dtype=np.uint32)[0]
    hi_trunc = np.frombuffer(
        np.uint32(bits & np.uint32(0xFFFFF000)).tobytes(), dtype=np.float32
    )[0]
    lo = np.float32(np.pi / 2 - np.float64(hi_trunc))
    return np.float32(hi_trunc), lo


_PIO2_C1, _PIO2_C2 = _split_pio2()

_S1 = np.float32(-1.6666654611e-01)
_S2 = np.float32(8.3321608736e-03)
_S3 = np.float32(-1.9515295891e-04)
_C1F = np.float32(4.1666667908e-02)
_C2F = np.float32(-1.3888889225e-03)
_C3F = np.float32(2.4760126784e-05)


def _sincos(x):
    nf = jax.lax.round(x * _TWO_OVER_PI,
                       jax.lax.RoundingMethod.TO_NEAREST_EVEN)
    q = nf.astype(jnp.int32)
    r = x - nf * _PIO2_C1
    r = r - nf * _PIO2_C2
    s = r * r
    ps = _S3
    ps = ps * s + _S2
    ps = ps * s + _S1
    ps = ps * s + np.float32(1.0)
    sinr = ps * r
    pc = _C3F
    pc = pc * s + _C2F
    pc = pc * s + _C1F
    cosr = (np.float32(1.0) - np.float32(0.5) * s) + (s * s) * pc
    swap = (q & 1) == 1
    cos_sel = jnp.where(swap, sinr, cosr)
    sin_sel = jnp.where(swap, cosr, sinr)
    sc = ((q + 1) & 2) << 30
    ss = (q & 2) << 30
    cos_out = jax.lax.bitcast_convert_type(
        jax.lax.bitcast_convert_type(cos_sel, jnp.int32) ^ sc, jnp.float32)
    sin_out = jax.lax.bitcast_convert_type(
        jax.lax.bitcast_convert_type(sin_sel, jnp.int32) ^ ss, jnp.float32)
    return cos_out, sin_out


def _reduce8(x):
    r = x.shape[0]
    acc = x[0:8, :]
    for i in range(8, r, 8):
        acc = acc + x[i:i + 8, :]
    return acc


def _sigreg_kernel(z_ref, a_ref, o_ref, acc_c, acc_s, an_s, *, nb, n_total, rc):
    ni = pl.program_id(1)

    @pl.when(ni == 0)
    def _():
        acc_c[...] = jnp.zeros_like(acc_c)
        acc_s[...] = jnp.zeros_like(acc_s)
        a0 = a_ref[...]
        inv = 1.0 / jnp.maximum(
            jnp.sqrt(jnp.sum(a0 * a0, axis=0, keepdims=True)), 1e-8)
        an_s[...] = a0 * inv

    an = an_s[...]

    bn = z_ref.shape[0]
    for c0 in range(0, bn, rc):
        zc = z_ref[c0:c0 + rc, :]
        proj = jnp.dot(zc, an, preferred_element_type=jnp.float32)
        theta = proj * _DT
        c1, s1 = _sincos(theta)
        tc = c1 + c1
        acc_c[0:8, :] += _reduce8(c1)
        acc_s[0:8, :] += _reduce8(s1)
        ckm, sk_m = c1, s1
        ck = tc * c1 - np.float32(1.0)
        sk = tc * s1
        acc_c[8:16, :] += _reduce8(ck)
        acc_s[8:16, :] += _reduce8(sk)
        for k in range(3, _NK + 1):
            ck, ckm = tc * ck - ckm, ck
            sk, sk_m = tc * sk - sk_m, sk
            r0 = (k - 1) * 8
            acc_c[r0:r0 + 8, :] += _reduce8(ck)
            acc_s[r0:r0 + 8, :] += _reduce8(sk)

    @pl.when(ni == nb - 1)
    def _():
        inv_n = np.float32(1.0 / n_total)
        cm = jnp.concatenate(
            [jnp.sum(acc_c[8 * k:8 * k + 8, :], axis=0, keepdims=True)
             for k in range(_NK)], axis=0) * inv_n
        sm = jnp.concatenate(
            [jnp.sum(acc_s[8 * k:8 * k + 8, :], axis=0, keepdims=True)
             for k in range(_NK)], axis=0) * inv_n
        k = jax.lax.broadcasted_iota(jnp.int32, (_NK, 1), 0).astype(jnp.float32) + 1.0
        t = k * _DT
        phi = jnp.exp(-0.5 * t * t)
        w = jnp.where(k == _NK, _DT, np.float32(2.0) * _DT)
        wphi = w * phi
        err = (cm - phi) ** 2 + sm ** 2
        stat = jnp.sum(err * wphi, axis=0) * np.float32(n_total)
        o_ref[0, :] = stat


def kernel(z, A):
    n, d = z.shape
    d2, p = A.shape
    assert d == d2
    bp = min(_BP, p)
    bn = min(_BN, n)
    rc = min(_RC, bn)
    nb = n // bn
    assert n % bn == 0 and p % bp == 0

    stat = pl.pallas_call(
        functools.partial(_sigreg_kernel, nb=nb, n_total=n, rc=rc),
        out_shape=jax.ShapeDtypeStruct((1, p), jnp.float32),
        grid=(p // bp, nb),
        in_specs=[
            pl.BlockSpec((bn, d), lambda pi, ni: (ni, 0)),
            pl.BlockSpec((d, bp), lambda pi, ni: (0, pi)),
        ],
        out_specs=pl.BlockSpec((1, bp), lambda pi, ni: (0, pi)),
        scratch_shapes=[
            pltpu.VMEM((8 * _NK, bp), jnp.float32),
            pltpu.VMEM((8 * _NK, bp), jnp.float32),
            pltpu.VMEM((d, bp), jnp.float32),
        ],
        compiler_params=pltpu.CompilerParams(
            dimension_semantics=("parallel", "arbitrary"),
        ),
        name="sigreg_ecf",
    )(z, A)
    return jnp.mean(stat)
